# baseline (speedup 1.0000x reference)
import functools

import jax
import jax.numpy as jnp
from jax import lax
from jax.experimental import pallas as pl
from jax.experimental.pallas import tpu as pltpu
from jax.experimental.pallas import tpu_sc as plsc


def _softmax_body(x_ref, o_ref):
    x = x_ref[...]
    m = jnp.max(x, axis=1, keepdims=True)
    e = jnp.exp(x - m)
    o_ref[...] = e / jnp.sum(e, axis=1, keepdims=True)


def _softmax_tc(x):
    return pl.pallas_call(
        _softmax_body,
        out_shape=jax.ShapeDtypeStruct(x.shape, x.dtype),
    )(x)


def _make_gather_sc(T, C, N, n_workers, chunk):
    n_chunks_per_w = N // (n_workers * chunk)
    per_w = N // n_workers
    mesh = plsc.VectorSubcoreMesh(core_axis_name="c", subcore_axis_name="s")
    nc = 2

    @functools.partial(
        pl.kernel,
        mesh=mesh,
        compiler_params=pltpu.CompilerParams(use_tc_tiling_on_sc=False),
        out_type=jax.ShapeDtypeStruct((N, C), jnp.float32),
        scratch_types=[
            pltpu.VMEM((n_chunks_per_w, chunk), jnp.int32),
            pltpu.VMEM((chunk, C), jnp.float32),
            pltpu.SemaphoreType.DMA,
        ],
    )
    def gather_kernel(probs_hbm, idx_hbm, out_hbm, idx_v, rows_v, sem):
        wid = lax.axis_index("s") * nc + lax.axis_index("c")
        base = wid * per_w
        pltpu.sync_copy(idx_hbm.at[pl.ds(wid * n_chunks_per_w, n_chunks_per_w)], idx_v)

        def body(c, carry):
            pltpu.async_copy(probs_hbm.at[idx_v.at[c]], rows_v, sem).wait()
            pltpu.sync_copy(rows_v, out_hbm.at[pl.ds(base + c * chunk, chunk)])
            return carry

        lax.fori_loop(0, n_chunks_per_w, body, 0)

    return gather_kernel


def kernel(inputs_hour, catid_time_matrix):
    B, S = inputs_hour.shape
    T, C = catid_time_matrix.shape
    N = B * S
    n_workers = 32
    chunk = 40
    assert N % (n_workers * chunk) == 0

    probs = _softmax_tc(catid_time_matrix)
    idx2d = inputs_hour.astype(jnp.int32).reshape(N // chunk, chunk)
    gather = _make_gather_sc(T, C, N, n_workers, chunk)
    out = gather(probs, idx2d)
    return out.reshape(B, S, C)


# speedup vs baseline: 1.0026x; 1.0026x over previous
import functools

import jax
import jax.numpy as jnp
from jax import lax
from jax.experimental import pallas as pl
from jax.experimental.pallas import tpu as pltpu
from jax.experimental.pallas import tpu_sc as plsc


def _softmax_body(x_ref, o_ref):
    x = x_ref[...]
    m = jnp.max(x, axis=1, keepdims=True)
    e = jnp.exp(x - m)
    o_ref[...] = e / jnp.sum(e, axis=1, keepdims=True)


def _softmax_tc(x):
    return pl.pallas_call(
        _softmax_body,
        out_shape=jax.ShapeDtypeStruct(x.shape, x.dtype),
    )(x)


def _make_gather_sc(T, C, N, n_workers, chunk):
    n_chunks_per_w = N // (n_workers * chunk)
    per_w = N // n_workers
    mesh = plsc.VectorSubcoreMesh(core_axis_name="c", subcore_axis_name="s")
    nc = 2

    assert n_chunks_per_w % 2 == 0 and n_chunks_per_w >= 4

    @functools.partial(
        pl.kernel,
        mesh=mesh,
        compiler_params=pltpu.CompilerParams(use_tc_tiling_on_sc=False),
        out_type=jax.ShapeDtypeStruct((N, C), jnp.float32),
        scratch_types=[
            pltpu.VMEM((n_chunks_per_w, chunk), jnp.int32),
            pltpu.VMEM((chunk, C), jnp.float32),
            pltpu.VMEM((chunk, C), jnp.float32),
            pltpu.SemaphoreType.DMA,
            pltpu.SemaphoreType.DMA,
        ],
    )
    def gather_kernel(probs_hbm, idx_hbm, out_hbm, idx_v, buf0, buf1, wsem0, wsem1):
        wid = lax.axis_index("s") * nc + lax.axis_index("c")
        base = wid * per_w
        bufs = (buf0, buf1)
        wsems = (wsem0, wsem1)
        pltpu.sync_copy(idx_hbm.at[pl.ds(wid * n_chunks_per_w, n_chunks_per_w)], idx_v)

        def step(cc, b, first):
            if not first:
                pltpu.make_async_copy(
                    bufs[b], out_hbm.at[pl.ds(base + (cc - 2) * chunk, chunk)],
                    wsems[b],
                ).wait()
            pltpu.async_copy(probs_hbm.at[idx_v.at[cc]], bufs[b], wsems[b]).wait()
            pltpu.async_copy(bufs[b], out_hbm.at[pl.ds(base + cc * chunk, chunk)], wsems[b])

        step(0, 0, True)
        step(1, 1, True)

        @pl.loop(2, n_chunks_per_w, step=2)
        def _(c):
            step(c, 0, False)
            step(c + 1, 1, False)

        for b, cc in ((0, n_chunks_per_w - 2), (1, n_chunks_per_w - 1)):
            pltpu.make_async_copy(
                bufs[b], out_hbm.at[pl.ds(base + cc * chunk, chunk)], wsems[b]
            ).wait()

    return gather_kernel


def kernel(inputs_hour, catid_time_matrix):
    B, S = inputs_hour.shape
    T, C = catid_time_matrix.shape
    N = B * S
    n_workers = 32
    chunk = 40
    assert N % (n_workers * chunk) == 0

    probs = _softmax_tc(catid_time_matrix)
    idx2d = inputs_hour.astype(jnp.int32).reshape(N // chunk, chunk)
    gather = _make_gather_sc(T, C, N, n_workers, chunk)
    out = gather(probs, idx2d)
    return out.reshape(B, S, C)
